# baseline (speedup 1.0000x reference)
import functools

import jax
import jax.numpy as jnp
from jax import lax
from jax.experimental import pallas as pl
from jax.experimental.pallas import tpu as pltpu
from jax.experimental.pallas import tpu_sc as plsc

_NC = 2
_NS = 16
_NW = _NC * _NS
_CH = 128
_G = 4


def _make_gather(V, D, R1, R2):
  r1 = R1 // _NW
  r2 = R2 // _NW
  assert r1 % _CH == 0 and r2 % _CH == 0

  mesh = plsc.VectorSubcoreMesh(core_axis_name="c", subcore_axis_name="s")

  @functools.partial(
      pl.kernel,
      out_type=[
          jax.ShapeDtypeStruct((R1, _G * D), jnp.float32),
          jax.ShapeDtypeStruct((R2, _G * D), jnp.float32),
      ],
      mesh=mesh,
      compiler_params=pltpu.CompilerParams(use_tc_tiling_on_sc=False),
      scratch_types=[
          pltpu.VMEM((r1,), jnp.int32),
          pltpu.VMEM((r2,), jnp.int32),
          pltpu.VMEM((_CH, D), jnp.float32),
          pltpu.SemaphoreType.DMA,
      ],
  )
  def gather(emb1, emb2, idx1, idx2, out1, out2, idx1_v, idx2_v, rows_v, sem):
    wid = lax.axis_index("s") * _NC + lax.axis_index("c")
    b1 = wid * r1
    b2 = wid * r2

    for j in range(_G):
      pltpu.sync_copy(idx1.at[j, pl.ds(b1, r1)], idx1_v)

      def body1(i, _):
        off = pl.multiple_of(i * _CH, _CH)
        pltpu.async_copy(emb1.at[idx1_v.at[pl.ds(off, _CH)]], rows_v, sem).wait()
        pltpu.sync_copy(rows_v,
                        out1.at[pl.ds(b1 + off, _CH), pl.ds(j * D, D)])
        return 0

      lax.fori_loop(0, r1 // _CH, body1, 0)

      pltpu.sync_copy(idx2.at[j, pl.ds(b2, r2)], idx2_v)

      def body2(i, _):
        off = pl.multiple_of(i * _CH, _CH)
        pltpu.async_copy(emb2.at[idx2_v.at[pl.ds(off, _CH)]], rows_v, sem).wait()
        pltpu.sync_copy(rows_v,
                        out2.at[pl.ds(b2 + off, _CH), pl.ds(j * D, D)])
        return 0

      lax.fori_loop(0, r2 // _CH, body2, 0)

  return gather


def _score_body(C, K1, CP, KP, GPB, eI_ref, eTV_ref, out_ref):
  LN = _G * 64

  @pl.when(pl.program_id(0) == 0)
  def _init():
    out_ref[...] = jnp.zeros_like(out_ref)

  rows_a = _G * CP

  def group(g, acc):
    tv4 = eTV_ref[pl.ds(g * KP, KP), :].astype(jnp.bfloat16)
    i4 = eI_ref[pl.ds(g * CP, CP), :].astype(jnp.bfloat16)
    lane = lax.broadcasted_iota(jnp.int32, (CP, LN), 1)
    a = jnp.concatenate(
        [jnp.where(lane // 64 == j, i4, jnp.bfloat16(0.0)) for j in range(_G)],
        axis=0)
    n = lax.dot_general(tv4, a, (((1,), (1,)), ((), ())),
                        preferred_element_type=jnp.float32)
    krow = lax.broadcasted_iota(jnp.int32, (KP, rows_a), 0)
    ccol = lax.broadcasted_iota(jnp.int32, (KP, rows_a), 1)
    cvalid = ccol % CP < C
    sc = jnp.where((krow == 0) & cvalid, n, 0.0)
    acc_s = jnp.sum(sc)
    e = jnp.where((krow >= 1) & (krow < K1), jnp.exp(n), 0.0)
    colsum = jnp.sum(e, axis=0, keepdims=True)
    nl = jnp.where(ccol[0:1] % CP < C, jnp.log(colsum), 0.0)
    acc_n = jnp.sum(nl)
    return acc + (acc_n - acc_s)

  p = lax.fori_loop(0, GPB, group, jnp.float32(0.0))
  out_ref[...] += jnp.reshape(p, (1, 1))


def kernel(inputs, targets, vocab_batches, emb1, emb2):
  B, C = inputs.shape
  K = vocab_batches.shape[1]
  V, D = emb1.shape
  K1 = K + 1
  KP = 56
  CP = 24
  NG = B // _G

  ip = jnp.pad(inputs.astype(jnp.int32), ((0, 0), (0, CP - C)))
  tvp = jnp.pad(
      jnp.concatenate(
          [targets.astype(jnp.int32), vocab_batches.astype(jnp.int32)],
          axis=1), ((0, 0), (0, KP - K1)))
  idx1 = ip.reshape(NG, _G, CP).transpose(1, 0, 2).reshape(_G, NG * CP)
  idx2 = tvp.reshape(NG, _G, KP).transpose(1, 0, 2).reshape(_G, NG * KP)

  eI4, eTV4 = _make_gather(V, D, NG * CP, NG * KP)(emb1, emb2, idx1, idx2)

  GPB = 64

  partial = pl.pallas_call(
      functools.partial(_score_body, C, K1, CP, KP, GPB),
      grid=(NG // GPB,),
      in_specs=[
          pl.BlockSpec((GPB * CP, _G * D), lambda i: (i, 0)),
          pl.BlockSpec((GPB * KP, _G * D), lambda i: (i, 0)),
      ],
      out_specs=pl.BlockSpec((1, 1), lambda i: (0, 0)),
      out_shape=jax.ShapeDtypeStruct((1, 1), jnp.float32),
  )(eI4, eTV4)

  return partial[0, 0] / jnp.float32(B * C)


# speedup vs baseline: 1.6319x; 1.6319x over previous
import functools

import jax
import jax.numpy as jnp
from jax import lax
from jax.experimental import pallas as pl
from jax.experimental.pallas import tpu as pltpu
from jax.experimental.pallas import tpu_sc as plsc

_NC = 2
_NS = 16
_NW = _NC * _NS
_CH = 128
_G = 4


def _make_gather(V, D, N1, N2):
  n1 = N1 // _NW
  n2 = N2 // _NW
  assert n1 % _CH == 0 and n2 % _CH == 0

  mesh = plsc.VectorSubcoreMesh(core_axis_name="c", subcore_axis_name="s")

  @functools.partial(
      pl.kernel,
      out_type=[
          jax.ShapeDtypeStruct((N1, D), jnp.float32),
          jax.ShapeDtypeStruct((N2, D), jnp.float32),
      ],
      mesh=mesh,
      compiler_params=pltpu.CompilerParams(use_tc_tiling_on_sc=False),
      scratch_types=[
          pltpu.VMEM((n1,), jnp.int32),
          pltpu.VMEM((n2,), jnp.int32),
          pltpu.VMEM((_CH, D), jnp.float32),
          pltpu.SemaphoreType.DMA,
      ],
  )
  def gather(emb1, emb2, idx1, idx2, out1, out2, idx1_v, idx2_v, rows_v, sem):
    wid = lax.axis_index("s") * _NC + lax.axis_index("c")
    b1 = wid * n1
    b2 = wid * n2
    pltpu.sync_copy(idx1.at[pl.ds(b1, n1)], idx1_v)
    pltpu.sync_copy(idx2.at[pl.ds(b2, n2)], idx2_v)

    def body1(i, _):
      off = pl.multiple_of(i * _CH, _CH)
      pltpu.async_copy(emb1.at[idx1_v.at[pl.ds(off, _CH)]], rows_v, sem).wait()
      pltpu.sync_copy(rows_v, out1.at[pl.ds(b1 + off, _CH)])
      return 0

    lax.fori_loop(0, n1 // _CH, body1, 0)

    def body2(i, _):
      off = pl.multiple_of(i * _CH, _CH)
      pltpu.async_copy(emb2.at[idx2_v.at[pl.ds(off, _CH)]], rows_v, sem).wait()
      pltpu.sync_copy(rows_v, out2.at[pl.ds(b2 + off, _CH)])
      return 0

    lax.fori_loop(0, n2 // _CH, body2, 0)

  return gather


def _score_body(C, K1, KP, GPB, eI_ref, eTV_ref, out_ref):
  RA = _G * C
  RX = _G * KP

  @pl.when(pl.program_id(0) == 0)
  def _init():
    out_ref[...] = jnp.zeros_like(out_ref)

  rr = lax.broadcasted_iota(jnp.int32, (RA, RX), 0)
  cc = lax.broadcasted_iota(jnp.int32, (RA, RX), 1)
  diag = (rr // C) == (cc // KP)
  kk = cc % KP
  exp_mask = diag & (kk >= 1) & (kk < K1)
  sc_mask = diag & (kk == 0)

  def group(g, acc):
    a = eI_ref[pl.ds(g * RA, RA), :].astype(jnp.bfloat16)
    x = eTV_ref[pl.ds(g * RX, RX), :].astype(jnp.bfloat16)
    p = lax.dot_general(a, x, (((1,), (1,)), ((), ())),
                        preferred_element_type=jnp.float32)
    acc_s = jnp.sum(jnp.where(sc_mask, p, 0.0))
    e = jnp.where(exp_mask, jnp.exp(p), 0.0)
    rowsum = jnp.sum(e, axis=1, keepdims=True)
    acc_n = jnp.sum(jnp.log(rowsum))
    return acc + (acc_n - acc_s)

  p = lax.fori_loop(0, GPB, group, jnp.float32(0.0))
  out_ref[...] += jnp.reshape(p, (1, 1))


def kernel(inputs, targets, vocab_batches, emb1, emb2):
  B, C = inputs.shape
  K = vocab_batches.shape[1]
  V, D = emb1.shape
  K1 = K + 1
  KP = 52

  idx1 = inputs.astype(jnp.int32).reshape(-1)
  idx2 = jnp.pad(
      jnp.concatenate(
          [targets.astype(jnp.int32), vocab_batches.astype(jnp.int32)],
          axis=1), ((0, 0), (0, KP - K1))).reshape(-1)

  eI, eTV = _make_gather(V, D, B * C, B * KP)(emb1, emb2, idx1, idx2)

  NG = B // _G
  GPB = 64

  partial = pl.pallas_call(
      functools.partial(_score_body, C, K1, KP, GPB),
      grid=(NG // GPB,),
      in_specs=[
          pl.BlockSpec((GPB * _G * C, D), lambda i: (i, 0)),
          pl.BlockSpec((GPB * _G * KP, D), lambda i: (i, 0)),
      ],
      out_specs=pl.BlockSpec((1, 1), lambda i: (0, 0)),
      out_shape=jax.ShapeDtypeStruct((1, 1), jnp.float32),
  )(eI, eTV)

  return partial[0, 0] / jnp.float32(B * C)


# speedup vs baseline: 2.5688x; 1.5741x over previous
import functools

import jax
import jax.numpy as jnp
from jax import lax
from jax.experimental import pallas as pl
from jax.experimental.pallas import tpu as pltpu
from jax.experimental.pallas import tpu_sc as plsc

_NC = 2
_NS = 16
_NW = _NC * _NS
_CH = 128
_G = 4


def _make_gather(V, D, N1, N2):
  n1 = N1 // _NW
  n2 = N2 // _NW
  assert n1 % _CH == 0 and n2 % _CH == 0

  mesh = plsc.VectorSubcoreMesh(core_axis_name="c", subcore_axis_name="s")

  @functools.partial(
      pl.kernel,
      out_type=[
          jax.ShapeDtypeStruct((N1, D), jnp.float32),
          jax.ShapeDtypeStruct((N2, D), jnp.float32),
      ],
      mesh=mesh,
      compiler_params=pltpu.CompilerParams(use_tc_tiling_on_sc=False),
      scratch_types=[
          pltpu.VMEM((n1,), jnp.int32),
          pltpu.VMEM((n2,), jnp.int32),
          pltpu.VMEM((_CH, D), jnp.float32),
          pltpu.SemaphoreType.DMA,
      ],
  )
  def gather(emb1, emb2, idx1, idx2, out1, out2, idx1_v, idx2_v, rows_v, sem):
    wid = lax.axis_index("s") * _NC + lax.axis_index("c")
    b1 = wid * n1
    b2 = wid * n2
    pltpu.sync_copy(idx1.at[pl.ds(b1, n1)], idx1_v)
    pltpu.sync_copy(idx2.at[pl.ds(b2, n2)], idx2_v)

    def body1(i, _):
      off = pl.multiple_of(i * _CH, _CH)
      pltpu.async_copy(emb1.at[idx1_v.at[pl.ds(off, _CH)]], rows_v, sem).wait()
      pltpu.sync_copy(rows_v, out1.at[pl.ds(b1 + off, _CH)])
      return 0

    lax.fori_loop(0, n1 // _CH, body1, 0)

    def body2(i, _):
      off = pl.multiple_of(i * _CH, _CH)
      pltpu.async_copy(emb2.at[idx2_v.at[pl.ds(off, _CH)]], rows_v, sem).wait()
      pltpu.sync_copy(rows_v, out2.at[pl.ds(b2 + off, _CH)])
      return 0

    lax.fori_loop(0, n2 // _CH, body2, 0)

  return gather


def _score_body(C, K1, KP, GPB, eI_ref, eTV_ref, out_ref):
  RA = _G * C
  RX = _G * KP

  @pl.when(pl.program_id(0) == 0)
  def _init():
    out_ref[...] = jnp.zeros_like(out_ref)

  rr = lax.broadcasted_iota(jnp.int32, (RA, RX), 0)
  cc = lax.broadcasted_iota(jnp.int32, (RA, RX), 1)
  diag = (rr // C) == (cc // KP)
  kk = cc % KP
  exp_mask = diag & (kk >= 1) & (kk < K1)
  sc_mask = diag & (kk == 0)

  UNROLL = 4

  def group(g4, acc):
    tot = None
    for u in range(UNROLL):
      g = g4 * UNROLL + u
      a = eI_ref[pl.ds(g * RA, RA), :].astype(jnp.bfloat16)
      x = eTV_ref[pl.ds(g * RX, RX), :].astype(jnp.bfloat16)
      p = lax.dot_general(a, x, (((1,), (1,)), ((), ())),
                          preferred_element_type=jnp.float32)
      acc_s = jnp.sum(jnp.where(sc_mask, p, 0.0))
      e = jnp.where(exp_mask, jnp.exp(p), 0.0)
      rowsum = jnp.sum(e, axis=1, keepdims=True)
      acc_n = jnp.sum(jnp.log(rowsum))
      part = acc_n - acc_s
      tot = part if tot is None else tot + part
    return acc + tot

  p = lax.fori_loop(0, GPB // UNROLL, group, jnp.float32(0.0))
  out_ref[...] += jnp.reshape(p, (1, 1))


def kernel(inputs, targets, vocab_batches, emb1, emb2):
  B, C = inputs.shape
  K = vocab_batches.shape[1]
  V, D = emb1.shape
  K1 = K + 1
  KP = 52

  idx1 = inputs.astype(jnp.int32).reshape(-1)
  idx2 = jnp.pad(
      jnp.concatenate(
          [targets.astype(jnp.int32), vocab_batches.astype(jnp.int32)],
          axis=1), ((0, 0), (0, KP - K1))).reshape(-1)

  eI, eTV = _make_gather(V, D, B * C, B * KP)(emb1, emb2, idx1, idx2)

  NG = B // _G
  GPB = 64

  partial = pl.pallas_call(
      functools.partial(_score_body, C, K1, KP, GPB),
      grid=(NG // GPB,),
      in_specs=[
          pl.BlockSpec((GPB * _G * C, D), lambda i: (i, 0)),
          pl.BlockSpec((GPB * _G * KP, D), lambda i: (i, 0)),
      ],
      out_specs=pl.BlockSpec((1, 1), lambda i: (0, 0)),
      out_shape=jax.ShapeDtypeStruct((1, 1), jnp.float32),
  )(eI, eTV)

  return partial[0, 0] / jnp.float32(B * C)


# speedup vs baseline: 2.6477x; 1.0307x over previous
import functools

import jax
import jax.numpy as jnp
from jax import lax
from jax.experimental import pallas as pl
from jax.experimental.pallas import tpu as pltpu
from jax.experimental.pallas import tpu_sc as plsc

_NC = 2
_NS = 16
_NW = _NC * _NS
_CH = 128
_G = 4


def _make_gather(V, D, N1, N2):
  n1 = N1 // _NW
  n2 = N2 // _NW
  assert n1 % _CH == 0 and n2 % _CH == 0

  mesh = plsc.VectorSubcoreMesh(core_axis_name="c", subcore_axis_name="s")

  @functools.partial(
      pl.kernel,
      out_type=[
          jax.ShapeDtypeStruct((N1, D), jnp.float32),
          jax.ShapeDtypeStruct((N2, D), jnp.float32),
      ],
      mesh=mesh,
      compiler_params=pltpu.CompilerParams(use_tc_tiling_on_sc=False),
      scratch_types=[
          pltpu.VMEM((n1,), jnp.int32),
          pltpu.VMEM((n2,), jnp.int32),
          pltpu.VMEM((_CH, D), jnp.float32),
          pltpu.SemaphoreType.DMA,
      ],
  )
  def gather(emb1, emb2, idx1, idx2, out1, out2, idx1_v, idx2_v, rows_v, sem):
    wid = lax.axis_index("s") * _NC + lax.axis_index("c")
    b1 = wid * n1
    b2 = wid * n2
    pltpu.sync_copy(idx1.at[pl.ds(b1, n1)], idx1_v)
    pltpu.sync_copy(idx2.at[pl.ds(b2, n2)], idx2_v)

    def body1(i, _):
      off = pl.multiple_of(i * _CH, _CH)
      pltpu.async_copy(emb1.at[idx1_v.at[pl.ds(off, _CH)]], rows_v, sem).wait()
      pltpu.sync_copy(rows_v, out1.at[pl.ds(b1 + off, _CH)])
      return 0

    lax.fori_loop(0, n1 // _CH, body1, 0)

    def body2(i, _):
      off = pl.multiple_of(i * _CH, _CH)
      pltpu.async_copy(emb2.at[idx2_v.at[pl.ds(off, _CH)]], rows_v, sem).wait()
      pltpu.sync_copy(rows_v, out2.at[pl.ds(b2 + off, _CH)])
      return 0

    lax.fori_loop(0, n2 // _CH, body2, 0)

  return gather


def _score_body(C, K1, KP, GPB, eI_ref, eTV_ref, out_ref):
  RA = _G * C
  RX = _G * KP

  @pl.when(pl.program_id(0) == 0)
  def _init():
    out_ref[...] = jnp.zeros_like(out_ref)

  rr = lax.broadcasted_iota(jnp.int32, (RX, RA), 0)
  cc = lax.broadcasted_iota(jnp.int32, (RX, RA), 1)
  diag = (rr // KP) == (cc // C)
  kk = rr % KP
  exp_mask = diag & (kk >= 1) & (kk < K1)
  sc_mask = diag & (kk == 0)

  UNROLL = 4

  def group(g4, acc):
    tot = None
    for u in range(UNROLL):
      g = g4 * UNROLL + u
      a = eI_ref[pl.ds(g * RA, RA), :].astype(jnp.bfloat16)
      x = eTV_ref[pl.ds(g * RX, RX), :].astype(jnp.bfloat16)
      p = lax.dot_general(x, a, (((1,), (1,)), ((), ())),
                          preferred_element_type=jnp.float32)
      acc_s = jnp.sum(jnp.where(sc_mask, p, 0.0))
      e = jnp.where(exp_mask, jnp.exp(p), 0.0)
      colsum = jnp.sum(e, axis=0, keepdims=True)
      acc_n = jnp.sum(jnp.log(colsum))
      part = acc_n - acc_s
      tot = part if tot is None else tot + part
    return acc + tot

  p = lax.fori_loop(0, GPB // UNROLL, group, jnp.float32(0.0))
  out_ref[...] += jnp.reshape(p, (1, 1))


def kernel(inputs, targets, vocab_batches, emb1, emb2):
  B, C = inputs.shape
  K = vocab_batches.shape[1]
  V, D = emb1.shape
  K1 = K + 1
  KP = 52

  idx1 = inputs.astype(jnp.int32).reshape(-1)
  idx2 = jnp.pad(
      jnp.concatenate(
          [targets.astype(jnp.int32), vocab_batches.astype(jnp.int32)],
          axis=1), ((0, 0), (0, KP - K1))).reshape(-1)

  eI, eTV = _make_gather(V, D, B * C, B * KP)(emb1, emb2, idx1, idx2)

  NG = B // _G
  GPB = 64

  partial = pl.pallas_call(
      functools.partial(_score_body, C, K1, KP, GPB),
      grid=(NG // GPB,),
      in_specs=[
          pl.BlockSpec((GPB * _G * C, D), lambda i: (i, 0)),
          pl.BlockSpec((GPB * _G * KP, D), lambda i: (i, 0)),
      ],
      out_specs=pl.BlockSpec((1, 1), lambda i: (0, 0)),
      out_shape=jax.ShapeDtypeStruct((1, 1), jnp.float32),
  )(eI, eTV)

  return partial[0, 0] / jnp.float32(B * C)


# speedup vs baseline: 3.3261x; 1.2562x over previous
import functools

import jax
import jax.numpy as jnp
from jax import lax
from jax.experimental import pallas as pl
from jax.experimental.pallas import tpu as pltpu
from jax.experimental.pallas import tpu_sc as plsc

_NC = 2
_NS = 16
_NW = _NC * _NS
_CH = 128
_G = 4


def _make_gather(V, D, N1, N2):
  n1 = N1 // _NW
  n2 = N2 // _NW
  assert n1 % _CH == 0 and n2 % _CH == 0

  mesh = plsc.VectorSubcoreMesh(core_axis_name="c", subcore_axis_name="s")

  @functools.partial(
      pl.kernel,
      out_type=[
          jax.ShapeDtypeStruct((N1, D), jnp.float32),
          jax.ShapeDtypeStruct((N2, D), jnp.float32),
      ],
      mesh=mesh,
      compiler_params=pltpu.CompilerParams(use_tc_tiling_on_sc=False),
      scratch_types=[
          pltpu.VMEM((n1,), jnp.int32),
          pltpu.VMEM((n2,), jnp.int32),
          pltpu.VMEM((_CH, D), jnp.float32),
          pltpu.SemaphoreType.DMA,
      ],
  )
  def gather(emb1, emb2, idx1, idx2, out1, out2, idx1_v, idx2_v, rows_v, sem):
    wid = lax.axis_index("s") * _NC + lax.axis_index("c")
    b1 = wid * n1
    b2 = wid * n2
    pltpu.sync_copy(idx1.at[pl.ds(b1, n1)], idx1_v)
    pltpu.sync_copy(idx2.at[pl.ds(b2, n2)], idx2_v)

    def body1(i, _):
      off = pl.multiple_of(i * _CH, _CH)
      pltpu.async_copy(emb1.at[idx1_v.at[pl.ds(off, _CH)]], rows_v, sem).wait()
      pltpu.sync_copy(rows_v, out1.at[pl.ds(b1 + off, _CH)])
      return 0

    lax.fori_loop(0, n1 // _CH, body1, 0)

    def body2(i, _):
      off = pl.multiple_of(i * _CH, _CH)
      pltpu.async_copy(emb2.at[idx2_v.at[pl.ds(off, _CH)]], rows_v, sem).wait()
      pltpu.sync_copy(rows_v, out2.at[pl.ds(b2 + off, _CH)])
      return 0

    lax.fori_loop(0, n2 // _CH, body2, 0)

  return gather


def _score_body(C, K1, KP, D, GPB, eI_ref, eTV_ref, out_ref):
  RA = _G * C
  RX = _G * KP
  HA = RA // 2
  HX = RX // 2

  @pl.when(pl.program_id(0) == 0)
  def _init():
    out_ref[...] = jnp.zeros_like(out_ref)

  qr = lax.broadcasted_iota(jnp.int32, (RX, RA), 0)
  qc = lax.broadcasted_iota(jnp.int32, (RX, RA), 1)
  lr = jnp.where(qr < HX, 2 * qr, 2 * (qr - HX) + 1)
  lc = jnp.where(qc < HA, 2 * qc, 2 * (qc - HA) + 1)
  diag = (lr // KP) == (lc // C)
  kk = lr % KP
  exp_mask = diag & (kk >= 1) & (kk < K1)
  sc_mask = diag & (kk == 0)

  UNROLL = 4

  def group(g4, acc):
    tot = None
    for u in range(UNROLL):
      g = g4 * UNROLL + u
      a2 = eI_ref[pl.ds(g * HA, HA), :].astype(jnp.bfloat16)
      x2 = eTV_ref[pl.ds(g * HX, HX), :].astype(jnp.bfloat16)
      a = jnp.concatenate([a2[:, :D], a2[:, D:]], axis=0)
      x = jnp.concatenate([x2[:, :D], x2[:, D:]], axis=0)
      p = lax.dot_general(x, a, (((1,), (1,)), ((), ())),
                          preferred_element_type=jnp.float32)
      acc_s = jnp.sum(jnp.where(sc_mask, p, 0.0))
      e = jnp.where(exp_mask, jnp.exp(p), 0.0)
      colsum = jnp.sum(e, axis=0, keepdims=True)
      acc_n = jnp.sum(jnp.log(colsum))
      part = acc_n - acc_s
      tot = part if tot is None else tot + part
    return acc + tot

  p = lax.fori_loop(0, GPB // UNROLL, group, jnp.float32(0.0))
  out_ref[...] += jnp.reshape(p, (1, 1))


def kernel(inputs, targets, vocab_batches, emb1, emb2):
  B, C = inputs.shape
  K = vocab_batches.shape[1]
  V, D = emb1.shape
  K1 = K + 1
  KP = 52

  idx1 = inputs.astype(jnp.int32).reshape(-1)
  idx2 = jnp.pad(
      jnp.concatenate(
          [targets.astype(jnp.int32), vocab_batches.astype(jnp.int32)],
          axis=1), ((0, 0), (0, KP - K1))).reshape(-1)

  eI, eTV = _make_gather(V, D, B * C, B * KP)(emb1, emb2, idx1, idx2)
  eI2 = eI.reshape(B * C // 2, 2 * D)
  eTV2 = eTV.reshape(B * KP // 2, 2 * D)

  NG = B // _G
  GPB = 64

  partial = pl.pallas_call(
      functools.partial(_score_body, C, K1, KP, D, GPB),
      grid=(NG // GPB,),
      in_specs=[
          pl.BlockSpec((GPB * _G * C // 2, 2 * D), lambda i: (i, 0)),
          pl.BlockSpec((GPB * _G * KP // 2, 2 * D), lambda i: (i, 0)),
      ],
      out_specs=pl.BlockSpec((1, 1), lambda i: (0, 0)),
      out_shape=jax.ShapeDtypeStruct((1, 1), jnp.float32),
  )(eI2, eTV2)

  return partial[0, 0] / jnp.float32(B * C)


# speedup vs baseline: 3.4164x; 1.0272x over previous
import functools

import jax
import jax.numpy as jnp
from jax import lax
from jax.experimental import pallas as pl
from jax.experimental.pallas import tpu as pltpu
from jax.experimental.pallas import tpu_sc as plsc

_NC = 2
_NS = 16
_NW = _NC * _NS
_CH = 128
_G = 4


def _make_gather(V, D, N1, N2):
  n1 = N1 // _NW
  n2 = N2 // _NW
  assert n1 % _CH == 0 and n2 % _CH == 0

  mesh = plsc.VectorSubcoreMesh(core_axis_name="c", subcore_axis_name="s")

  NB = 4

  @functools.partial(
      pl.kernel,
      out_type=[
          jax.ShapeDtypeStruct((N1, D), jnp.float32),
          jax.ShapeDtypeStruct((N2, D), jnp.float32),
      ],
      mesh=mesh,
      compiler_params=pltpu.CompilerParams(use_tc_tiling_on_sc=False),
      scratch_types=(
          [pltpu.VMEM((n1,), jnp.int32), pltpu.VMEM((n2,), jnp.int32)]
          + [pltpu.VMEM((_CH, D), jnp.float32) for _ in range(NB)]
          + [pltpu.SemaphoreType.DMA for _ in range(NB)]
      ),
  )
  def gather(emb1, emb2, idx1, idx2, out1, out2, idx1_v, idx2_v, *bufs_sems):
    bufs = bufs_sems[:NB]
    sems = bufs_sems[NB:]
    wid = lax.axis_index("s") * _NC + lax.axis_index("c")
    b1 = wid * n1
    b2 = wid * n2
    pltpu.sync_copy(idx1.at[pl.ds(b1, n1)], idx1_v)
    pltpu.sync_copy(idx2.at[pl.ds(b2, n2)], idx2_v)

    def make_body(emb, idx_v, out, base):
      def body(i4, _):
        cps = []
        for b in range(NB):
          off = pl.multiple_of((i4 * NB + b) * _CH, _CH)
          cps.append(
              pltpu.async_copy(emb.at[idx_v.at[pl.ds(off, _CH)]],
                               bufs[b], sems[b]))
        for b in range(NB):
          off = pl.multiple_of((i4 * NB + b) * _CH, _CH)
          cps[b].wait()
          pltpu.sync_copy(bufs[b], out.at[pl.ds(base + off, _CH)])
        return 0

      return body

    lax.fori_loop(0, n1 // (_CH * NB), make_body(emb1, idx1_v, out1, b1), 0)
    lax.fori_loop(0, n2 // (_CH * NB), make_body(emb2, idx2_v, out2, b2), 0)

  return gather


def _score_body(C, K1, KP, D, GPB, eI_ref, eTV_ref, out_ref):
  RA = _G * C
  RX = _G * KP
  HA = RA // 2
  HX = RX // 2

  @pl.when(pl.program_id(0) == 0)
  def _init():
    out_ref[...] = jnp.zeros_like(out_ref)

  qr = lax.broadcasted_iota(jnp.int32, (RX, RA), 0)
  qc = lax.broadcasted_iota(jnp.int32, (RX, RA), 1)
  lr = jnp.where(qr < HX, 2 * qr, 2 * (qr - HX) + 1)
  lc = jnp.where(qc < HA, 2 * qc, 2 * (qc - HA) + 1)
  diag = (lr // KP) == (lc // C)
  kk = lr % KP
  exp_mask = diag & (kk >= 1) & (kk < K1)
  sc_mask = diag & (kk == 0)

  UNROLL = 4

  def group(g4, acc):
    tot = None
    for u in range(UNROLL):
      g = g4 * UNROLL + u
      a2 = eI_ref[pl.ds(g * HA, HA), :].astype(jnp.bfloat16)
      x2 = eTV_ref[pl.ds(g * HX, HX), :].astype(jnp.bfloat16)
      a = jnp.concatenate([a2[:, :D], a2[:, D:]], axis=0)
      x = jnp.concatenate([x2[:, :D], x2[:, D:]], axis=0)
      p = lax.dot_general(x, a, (((1,), (1,)), ((), ())),
                          preferred_element_type=jnp.float32)
      acc_s = jnp.sum(jnp.where(sc_mask, p, 0.0))
      e = jnp.where(exp_mask, jnp.exp(p), 0.0)
      colsum = jnp.sum(e, axis=0, keepdims=True)
      acc_n = jnp.sum(jnp.log(colsum))
      part = acc_n - acc_s
      tot = part if tot is None else tot + part
    return acc + tot

  p = lax.fori_loop(0, GPB // UNROLL, group, jnp.float32(0.0))
  out_ref[...] += jnp.reshape(p, (1, 1))


def kernel(inputs, targets, vocab_batches, emb1, emb2):
  B, C = inputs.shape
  K = vocab_batches.shape[1]
  V, D = emb1.shape
  K1 = K + 1
  KP = 52

  idx1 = inputs.astype(jnp.int32).reshape(-1)
  idx2 = jnp.pad(
      jnp.concatenate(
          [targets.astype(jnp.int32), vocab_batches.astype(jnp.int32)],
          axis=1), ((0, 0), (0, KP - K1))).reshape(-1)

  eI, eTV = _make_gather(V, D, B * C, B * KP)(emb1, emb2, idx1, idx2)
  eI2 = eI.reshape(B * C // 2, 2 * D)
  eTV2 = eTV.reshape(B * KP // 2, 2 * D)

  NG = B // _G
  GPB = 64

  partial = pl.pallas_call(
      functools.partial(_score_body, C, K1, KP, D, GPB),
      grid=(NG // GPB,),
      in_specs=[
          pl.BlockSpec((GPB * _G * C // 2, 2 * D), lambda i: (i, 0)),
          pl.BlockSpec((GPB * _G * KP // 2, 2 * D), lambda i: (i, 0)),
      ],
      out_specs=pl.BlockSpec((1, 1), lambda i: (0, 0)),
      out_shape=jax.ShapeDtypeStruct((1, 1), jnp.float32),
  )(eI2, eTV2)

  return partial[0, 0] / jnp.float32(B * C)


# speedup vs baseline: 3.4312x; 1.0043x over previous
import functools

import jax
import jax.numpy as jnp
from jax import lax
from jax.experimental import pallas as pl
from jax.experimental.pallas import tpu as pltpu
from jax.experimental.pallas import tpu_sc as plsc

_NC = 2
_NS = 16
_NW = _NC * _NS
_CH = 128
_G = 4


def _make_gather(V, D, N1, N2):
  n1 = N1 // _NW
  n2 = N2 // _NW
  assert n1 % _CH == 0 and n2 % _CH == 0

  mesh = plsc.VectorSubcoreMesh(core_axis_name="c", subcore_axis_name="s")

  NB = 8

  @functools.partial(
      pl.kernel,
      out_type=[
          jax.ShapeDtypeStruct((N1, D), jnp.float32),
          jax.ShapeDtypeStruct((N2, D), jnp.float32),
      ],
      mesh=mesh,
      compiler_params=pltpu.CompilerParams(use_tc_tiling_on_sc=False),
      scratch_types=(
          [pltpu.VMEM((n1,), jnp.int32), pltpu.VMEM((n2,), jnp.int32)]
          + [pltpu.VMEM((_CH, D), jnp.float32) for _ in range(NB)]
          + [pltpu.SemaphoreType.DMA for _ in range(NB)]
      ),
  )
  def gather(emb1, emb2, idx1, idx2, out1, out2, idx1_v, idx2_v, *bufs_sems):
    bufs = bufs_sems[:NB]
    sems = bufs_sems[NB:]
    wid = lax.axis_index("s") * _NC + lax.axis_index("c")
    b1 = wid * n1
    b2 = wid * n2
    pltpu.sync_copy(idx1.at[pl.ds(b1, n1)], idx1_v)
    pltpu.sync_copy(idx2.at[pl.ds(b2, n2)], idx2_v)

    def make_body(emb, idx_v, out, base):
      def body(i4, _):
        cps = []
        for b in range(NB):
          off = pl.multiple_of((i4 * NB + b) * _CH, _CH)
          cps.append(
              pltpu.async_copy(emb.at[idx_v.at[pl.ds(off, _CH)]],
                               bufs[b], sems[b]))
        for b in range(NB):
          off = pl.multiple_of((i4 * NB + b) * _CH, _CH)
          cps[b].wait()
          pltpu.sync_copy(bufs[b], out.at[pl.ds(base + off, _CH)])
        return 0

      return body

    lax.fori_loop(0, n1 // (_CH * NB), make_body(emb1, idx1_v, out1, b1), 0)
    lax.fori_loop(0, n2 // (_CH * NB), make_body(emb2, idx2_v, out2, b2), 0)

  return gather


def _score_body(C, K1, KP, D, GPB, eI_ref, eTV_ref, out_ref):
  RA = _G * C
  RX = _G * KP
  HA = RA // 2
  HX = RX // 2

  @pl.when(pl.program_id(0) == 0)
  def _init():
    out_ref[...] = jnp.zeros_like(out_ref)

  qr = lax.broadcasted_iota(jnp.int32, (RX, RA), 0)
  qc = lax.broadcasted_iota(jnp.int32, (RX, RA), 1)
  lr = jnp.where(qr < HX, 2 * qr, 2 * (qr - HX) + 1)
  lc = jnp.where(qc < HA, 2 * qc, 2 * (qc - HA) + 1)
  diag = (lr // KP) == (lc // C)
  kk = lr % KP
  exp_mask = diag & (kk >= 1) & (kk < K1)
  sc_mask = diag & (kk == 0)

  UNROLL = 4

  def group(g4, acc):
    tot = None
    for u in range(UNROLL):
      g = g4 * UNROLL + u
      a2 = eI_ref[pl.ds(g * HA, HA), :].astype(jnp.bfloat16)
      x2 = eTV_ref[pl.ds(g * HX, HX), :].astype(jnp.bfloat16)
      a = jnp.concatenate([a2[:, :D], a2[:, D:]], axis=0)
      x = jnp.concatenate([x2[:, :D], x2[:, D:]], axis=0)
      p = lax.dot_general(x, a, (((1,), (1,)), ((), ())),
                          preferred_element_type=jnp.float32)
      acc_s = jnp.sum(jnp.where(sc_mask, p, 0.0))
      e = jnp.where(exp_mask, jnp.exp(p), 0.0)
      colsum = jnp.sum(e, axis=0, keepdims=True)
      acc_n = jnp.sum(jnp.log(colsum))
      part = acc_n - acc_s
      tot = part if tot is None else tot + part
    return acc + tot

  p = lax.fori_loop(0, GPB // UNROLL, group, jnp.float32(0.0))
  out_ref[...] += jnp.reshape(p, (1, 1))


def kernel(inputs, targets, vocab_batches, emb1, emb2):
  B, C = inputs.shape
  K = vocab_batches.shape[1]
  V, D = emb1.shape
  K1 = K + 1
  KP = 52

  idx1 = inputs.astype(jnp.int32).reshape(-1)
  idx2 = jnp.pad(
      jnp.concatenate(
          [targets.astype(jnp.int32), vocab_batches.astype(jnp.int32)],
          axis=1), ((0, 0), (0, KP - K1))).reshape(-1)

  eI, eTV = _make_gather(V, D, B * C, B * KP)(emb1, emb2, idx1, idx2)
  eI2 = eI.reshape(B * C // 2, 2 * D)
  eTV2 = eTV.reshape(B * KP // 2, 2 * D)

  NG = B // _G
  GPB = 64

  partial = pl.pallas_call(
      functools.partial(_score_body, C, K1, KP, D, GPB),
      grid=(NG // GPB,),
      in_specs=[
          pl.BlockSpec((GPB * _G * C // 2, 2 * D), lambda i: (i, 0)),
          pl.BlockSpec((GPB * _G * KP // 2, 2 * D), lambda i: (i, 0)),
      ],
      out_specs=pl.BlockSpec((1, 1), lambda i: (0, 0)),
      out_shape=jax.ShapeDtypeStruct((1, 1), jnp.float32),
  )(eI2, eTV2)

  return partial[0, 0] / jnp.float32(B * C)
